# baseline (speedup 1.0000x reference)
import functools

import jax
import jax.numpy as jnp
from jax import lax
from jax.experimental import pallas as pl
from jax.experimental.pallas import tpu as pltpu
from jax.experimental.pallas import tpu_sc as plsc

_TOKENS = 4 * 8192
_CH = 1024

_SC_TOKENS = 8192
_TC_TOKENS = _TOKENS - _SC_TOKENS

_TC_BLK = 512
_TC_DATA_STEPS = _TC_TOKENS // _TC_BLK
_TC_STEPS = _TOKENS // _TC_BLK

_SC_BLK = 16
_SC_BLK_OFF = _TC_TOKENS // _SC_BLK
_NC, _NS, _L = 2, 16, 16


def _tc_kernel(p_ref, t_ref, m_ref, sq_ref, cnt_ref):
    i = pl.program_id(0)

    @pl.when(i == 0)
    def _init():
        sq_ref[0, 0] = 0.0
        cnt_ref[0, 0] = 0.0

    m = m_ref[0, 0]
    cnt_ref[0, 0] += jnp.sum(m)

    @pl.when(i < _TC_DATA_STEPS)
    def _data():
        d = p_ref[...] - t_ref[...]
        row_sq = jnp.sum(d * d, axis=1)
        sq_ref[0, 0] += jnp.sum(row_sq * m)


def _tc_call(pred, tgt, m_full):
    return pl.pallas_call(
        _tc_kernel,
        grid=(_TC_STEPS,),
        in_specs=[
            pl.BlockSpec((_TC_BLK, _CH),
                         lambda i: (jnp.minimum(i, _TC_DATA_STEPS - 1), 0)),
            pl.BlockSpec((_TC_BLK, _CH),
                         lambda i: (jnp.minimum(i, _TC_DATA_STEPS - 1), 0)),
            pl.BlockSpec((1, 1, _TC_BLK), lambda i: (i, 0, 0)),
        ],
        out_specs=[
            pl.BlockSpec(memory_space=pltpu.SMEM),
            pl.BlockSpec(memory_space=pltpu.SMEM),
        ],
        out_shape=[
            jax.ShapeDtypeStruct((1, 1), jnp.float32),
            jax.ShapeDtypeStruct((1, 1), jnp.float32),
        ],
    )(pred, tgt, m_full)


def _sc_call(pred, tgt, m_exp):
    mesh = plsc.VectorSubcoreMesh(core_axis_name="c", subcore_axis_name="s")

    @functools.partial(
        pl.kernel,
        mesh=mesh,
        out_type=jax.ShapeDtypeStruct((_NC, _NS, _L), jnp.float32),
        scratch_types=[pltpu.VMEM((_L,), jnp.float32)],
    )
    def sc_kernel(p_hbm, t_hbm, m_hbm, o_hbm, acc_ref):
        c = lax.axis_index("c")
        s = lax.axis_index("s")
        acc_ref[...] = jnp.zeros((_L,), jnp.float32)

        def body(p_vmem, t_vmem, m_vmem):
            @pl.loop(0, _SC_BLK)
            def _row(r):
                def inner(c0, tmp):
                    sl = pl.ds(c0 * _L, _L)
                    d = p_vmem[r, sl] - t_vmem[r, sl]
                    return tmp + d * d

                tmp = lax.fori_loop(0, _CH // _L, inner,
                                    jnp.zeros((_L,), jnp.float32))
                acc_ref[...] += tmp * m_vmem[r]

        pltpu.emit_pipeline(
            body,
            grid=(_SC_TOKENS // _SC_BLK,),
            in_specs=[
                pl.BlockSpec(block_shape=(_SC_BLK, _CH),
                             index_map=lambda i: (i + _SC_BLK_OFF, 0)),
                pl.BlockSpec(block_shape=(_SC_BLK, _CH),
                             index_map=lambda i: (i + _SC_BLK_OFF, 0)),
                pl.BlockSpec(block_shape=(_SC_BLK, _L),
                             index_map=lambda i: (i, 0)),
            ],
            core_axis_name=("c", "s"),
            dimension_semantics=(pltpu.PARALLEL,),
        )(p_hbm, t_hbm, m_hbm)

        pltpu.sync_copy(acc_ref, o_hbm.at[c, s])

    return sc_kernel(pred, tgt, m_exp)


def kernel(predicted, target, mask):
    tgt_dim = target.shape[-1]
    pred = predicted[..., :tgt_dim].reshape(_TOKENS, _CH)
    tgt = target.reshape(_TOKENS, _CH)
    m_f32 = mask.reshape(_TOKENS).astype(jnp.float32)
    m_full = m_f32.reshape(_TC_STEPS, 1, _TC_BLK)
    m_exp = jnp.broadcast_to(m_f32[_TC_TOKENS:, None], (_SC_TOKENS, _L))

    sq_tc, cnt = _tc_call(pred, tgt, m_full)
    sc_part = _sc_call(pred, tgt, m_exp)

    total_sq = sq_tc[0, 0] + jnp.sum(sc_part)
    n = cnt[0, 0] * _CH
    return total_sq / jnp.maximum(n, 1.0)
